# Initial kernel scaffold; baseline (speedup 1.0000x reference)
import jax
import jax.numpy as jnp
from jax.experimental import pallas as pl


def kernel(x, edge_index, W1, b1, W2, b2):
    raise NotImplementedError("write your pallas kernel here")


# speedup vs baseline: 22.0663x; 22.0663x over previous
import functools

import jax
import jax.numpy as jnp
from jax import lax
from jax.experimental import pallas as pl
from jax.experimental.pallas import tpu as pltpu
from jax.experimental.pallas import tpu_sc as plsc

N = 10000
E = 320000
DF = 128
D1 = 16
D2 = 40
D2P = 48

NC = 2
NS = 16
NW = NC * NS

N_PAD = 10240
NPT = N_PAD // NS
CH = 128
E_PW = 10240
NCHK = E_PW // CH
E_PAD = E_PW * NW


def _flat_worker_id():
    return lax.axis_index("s") * NC + lax.axis_index("c")


def _sc_deg(col2):
    mesh = plsc.VectorSubcoreMesh(core_axis_name="c", subcore_axis_name="s")

    @functools.partial(
        pl.kernel,
        out_type=jax.ShapeDtypeStruct((NW, N_PAD), jnp.float32),
        mesh=mesh,
        compiler_params=pltpu.CompilerParams(needs_layout_passes=False),
        scratch_types=[
            pltpu.VMEM((E_PW,), jnp.int32),
            pltpu.VMEM((N_PAD,), jnp.float32),
        ],
    )
    def kd(col_hbm, out_hbm, col_v, acc_v):
        w = _flat_worker_id()
        pltpu.sync_copy(col_hbm.at[w], col_v)
        zeros = jnp.zeros((16,), jnp.float32)
        ones = jnp.ones((16,), jnp.float32)

        def zbody(i, carry):
            acc_v[pl.ds(i * 16, 16)] = zeros
            return carry

        lax.fori_loop(0, N_PAD // 16, zbody, 0)

        def body(i, carry):
            idx = col_v[pl.ds(i * 16, 16)]
            plsc.addupdate_scatter(acc_v, [idx], ones)
            return carry

        lax.fori_loop(0, E_PW // 16, body, 0)
        pltpu.sync_copy(acc_v, out_hbm.at[w])

    return kd(col2)


def _make_sc_scatter(D):
    mesh = plsc.VectorSubcoreMesh(core_axis_name="c", subcore_axis_name="s")

    @functools.partial(
        pl.kernel,
        out_type=jax.ShapeDtypeStruct((NC, N_PAD, D), jnp.float32),
        mesh=mesh,
        compiler_params=pltpu.CompilerParams(use_tc_tiling_on_sc=False),
        scratch_types=[
            pltpu.VMEM((NCHK, CH), jnp.int32),
            pltpu.VMEM((NCHK, CH), jnp.int32),
            pltpu.VMEM((CH, D), jnp.float32),
            pltpu.VMEM_SHARED((N_PAD, D), jnp.float32),
            pltpu.SemaphoreType.DMA,
        ],
    )
    def ks(g_hbm, row_hbm, col_hbm, zeros_hbm, out_hbm, row_v, col_v, rows_v,
           acc_sh, sem):
        c = lax.axis_index("c")
        s = lax.axis_index("s")
        w = s * NC + c
        pltpu.sync_copy(zeros_hbm.at[pl.ds(s * NPT, NPT)],
                        acc_sh.at[pl.ds(s * NPT, NPT)])
        pltpu.sync_copy(row_hbm.at[w], row_v)
        pltpu.sync_copy(col_hbm.at[w], col_v)
        plsc.subcore_barrier()

        def body(j, carry):
            pltpu.async_copy(g_hbm.at[row_v.at[j]], rows_v, sem).wait()
            pltpu.sync_copy(rows_v, acc_sh.at[col_v.at[j]], add=True)
            return carry

        lax.fori_loop(0, NCHK, body, 0)
        plsc.subcore_barrier()
        pltpu.sync_copy(acc_sh.at[pl.ds(s * NPT, NPT)],
                        out_hbm.at[c, pl.ds(s * NPT, NPT)])

    return ks


_sc_scatter16 = _make_sc_scatter(D1)
_sc_scatter48 = _make_sc_scatter(D2P)


def _tc_deg(degp):
    def body(degp_ref, dinv_ref):
        deg = jnp.sum(degp_ref[...], axis=0, keepdims=True) + 1.0
        dinv_ref[...] = lax.rsqrt(deg)

    return pl.pallas_call(
        body, out_shape=jax.ShapeDtypeStruct((1, N_PAD), jnp.float32))(degp)


def _tc1(x_pad, W1, dinv_col):
    def body(x_ref, w_ref, dv_ref, g_ref):
        h = jnp.dot(x_ref[...], w_ref[...], preferred_element_type=jnp.float32)
        g_ref[...] = h * dv_ref[...]

    return pl.pallas_call(
        body, out_shape=jax.ShapeDtypeStruct((N_PAD, D1), jnp.float32))(
            x_pad, W1, dinv_col)


def _tc2(S1, g1, dinv_col, b1_row, W2p):
    def body(s_ref, g1_ref, dv_ref, b1_ref, w2_ref, g2_ref):
        agg = (s_ref[0] + s_ref[1] + g1_ref[...]) * dv_ref[...]
        a1 = jnp.maximum(agg + b1_ref[...], 0.0)
        h2 = jnp.dot(a1, w2_ref[...], preferred_element_type=jnp.float32)
        g2_ref[...] = h2 * dv_ref[...]

    return pl.pallas_call(
        body, out_shape=jax.ShapeDtypeStruct((N_PAD, D2P), jnp.float32))(
            S1, g1, dinv_col, b1_row, W2p)


def _tc3(S2, g2, dinv_col, b2_row):
    def body(s_ref, g2_ref, dv_ref, b2_ref, out_ref):
        agg = (s_ref[0] + s_ref[1] + g2_ref[...]) * dv_ref[...]
        z = agg[:, :D2] + b2_ref[...]
        m = jnp.max(z, axis=1, keepdims=True)
        lse = jnp.log(jnp.sum(jnp.exp(z - m), axis=1, keepdims=True)) + m
        out_ref[...] = z - lse

    return pl.pallas_call(
        body, out_shape=jax.ShapeDtypeStruct((N_PAD, D2), jnp.float32))(
            S2, g2, dinv_col, b2_row)


def kernel(x, edge_index, W1, b1, W2, b2):
    ei = edge_index.astype(jnp.int32)
    pad = jnp.full((E_PAD - E,), N, jnp.int32)
    row = jnp.concatenate([ei[0], pad])
    col = jnp.concatenate([ei[1], pad])
    row3 = row.reshape(NW, NCHK, CH)
    col3 = col.reshape(NW, NCHK, CH)
    col2 = col.reshape(NW, E_PW)
    x_pad = jnp.pad(x, ((0, N_PAD - N), (0, 0)))
    W2p = jnp.pad(W2, ((0, 0), (0, D2P - D2)))
    zeros16 = jnp.zeros((N_PAD, D1), jnp.float32)
    zeros48 = jnp.zeros((N_PAD, D2P), jnp.float32)

    degp = _sc_deg(col2)
    dinv_row = _tc_deg(degp)
    dinv_col = dinv_row.reshape(N_PAD, 1)
    g1 = _tc1(x_pad, W1, dinv_col)
    S1 = _sc_scatter16(g1, row3, col3, zeros16)
    g2 = _tc2(S1, g1, dinv_col, b1.reshape(1, D1), W2p)
    S2 = _sc_scatter48(g2, row3, col3, zeros48)
    out = _tc3(S2, g2, dinv_col, b2.reshape(1, D2))
    return out[:N]


# speedup vs baseline: 27.3533x; 1.2396x over previous
import functools

import jax
import jax.numpy as jnp
from jax import lax
from jax.experimental import pallas as pl
from jax.experimental.pallas import tpu as pltpu
from jax.experimental.pallas import tpu_sc as plsc

N = 10000
E = 320000
DF = 128
D1 = 16
D2 = 40
D2P = 48

NC = 2
NS = 16
NW = NC * NS

N_PAD = 10240
NPT = N_PAD // NS
CH = 128
E_PW = 10240
NCHK = E_PW // CH
E_PAD = E_PW * NW


def _flat_worker_id():
    return lax.axis_index("s") * NC + lax.axis_index("c")


def _sc_deg(col2):
    mesh = plsc.VectorSubcoreMesh(core_axis_name="c", subcore_axis_name="s")

    @functools.partial(
        pl.kernel,
        out_type=jax.ShapeDtypeStruct((NW, N_PAD), jnp.float32),
        mesh=mesh,
        compiler_params=pltpu.CompilerParams(needs_layout_passes=False),
        scratch_types=[
            pltpu.VMEM((E_PW,), jnp.int32),
            pltpu.VMEM((N_PAD,), jnp.float32),
        ],
    )
    def kd(col_hbm, out_hbm, col_v, acc_v):
        w = _flat_worker_id()
        pltpu.sync_copy(col_hbm.at[w], col_v)
        zeros = jnp.zeros((16,), jnp.float32)
        ones = jnp.ones((16,), jnp.float32)

        def zbody(i, carry):
            acc_v[pl.ds(i * 16, 16)] = zeros
            return carry

        lax.fori_loop(0, N_PAD // 16, zbody, 0)

        def body(i, carry):
            idx = col_v[pl.ds(i * 16, 16)]
            plsc.addupdate_scatter(acc_v, [idx], ones)
            return carry

        lax.fori_loop(0, E_PW // 16, body, 0)
        pltpu.sync_copy(acc_v, out_hbm.at[w])

    return kd(col2)


def _make_sc_scatter(D):
    mesh = plsc.VectorSubcoreMesh(core_axis_name="c", subcore_axis_name="s")

    @functools.partial(
        pl.kernel,
        out_type=jax.ShapeDtypeStruct((NC, N_PAD, D), jnp.float32),
        mesh=mesh,
        compiler_params=pltpu.CompilerParams(use_tc_tiling_on_sc=False),
        scratch_types=[
            pltpu.VMEM((NCHK, CH), jnp.int32),
            pltpu.VMEM((NCHK, CH), jnp.int32),
            pltpu.VMEM((CH, D), jnp.float32),
            pltpu.VMEM((CH, D), jnp.float32),
            pltpu.VMEM_SHARED((N_PAD, D), jnp.float32),
            pltpu.SemaphoreType.DMA,
            pltpu.SemaphoreType.DMA,
        ],
    )
    def ks(g_hbm, row_hbm, col_hbm, zeros_hbm, out_hbm, row_v, col_v, rows_a,
           rows_b, acc_sh, sem_a, sem_b):
        c = lax.axis_index("c")
        s = lax.axis_index("s")
        w = s * NC + c
        pltpu.sync_copy(zeros_hbm.at[pl.ds(s * NPT, NPT)],
                        acc_sh.at[pl.ds(s * NPT, NPT)])
        pltpu.sync_copy(row_hbm.at[w], row_v)
        pltpu.sync_copy(col_hbm.at[w], col_v)
        plsc.subcore_barrier()

        pltpu.async_copy(g_hbm.at[row_v.at[0]], rows_a, sem_a)

        def body(t, carry):
            j0 = 2 * t
            pltpu.async_copy(g_hbm.at[row_v.at[j0 + 1]], rows_b, sem_b)
            pltpu.make_async_copy(g_hbm.at[row_v.at[j0]], rows_a, sem_a).wait()
            pltpu.sync_copy(rows_a, acc_sh.at[col_v.at[j0]], add=True)

            @pl.when(t < NCHK // 2 - 1)
            def _():
                pltpu.async_copy(g_hbm.at[row_v.at[j0 + 2]], rows_a, sem_a)

            pltpu.make_async_copy(
                g_hbm.at[row_v.at[j0 + 1]], rows_b, sem_b).wait()
            pltpu.sync_copy(rows_b, acc_sh.at[col_v.at[j0 + 1]], add=True)
            return carry

        lax.fori_loop(0, NCHK // 2, body, 0)
        plsc.subcore_barrier()
        pltpu.sync_copy(acc_sh.at[pl.ds(s * NPT, NPT)],
                        out_hbm.at[c, pl.ds(s * NPT, NPT)])

    return ks


_sc_scatter16 = _make_sc_scatter(D1)
_sc_scatter48 = _make_sc_scatter(D2P)


def _tc_deg(degp):
    def body(degp_ref, dinv_ref):
        deg = jnp.sum(degp_ref[...], axis=0, keepdims=True) + 1.0
        dinv_ref[...] = lax.rsqrt(deg)

    return pl.pallas_call(
        body, out_shape=jax.ShapeDtypeStruct((1, N_PAD), jnp.float32))(degp)


def _tc1(x_pad, W1, dinv_col):
    def body(x_ref, w_ref, dv_ref, g_ref):
        h = jnp.dot(x_ref[...], w_ref[...], preferred_element_type=jnp.float32)
        g_ref[...] = h * dv_ref[...]

    return pl.pallas_call(
        body, out_shape=jax.ShapeDtypeStruct((N_PAD, D1), jnp.float32))(
            x_pad, W1, dinv_col)


def _tc2(S1, g1, dinv_col, b1_row, W2p):
    def body(s_ref, g1_ref, dv_ref, b1_ref, w2_ref, g2_ref):
        agg = (s_ref[0] + s_ref[1] + g1_ref[...]) * dv_ref[...]
        a1 = jnp.maximum(agg + b1_ref[...], 0.0)
        h2 = jnp.dot(a1, w2_ref[...], preferred_element_type=jnp.float32)
        g2_ref[...] = h2 * dv_ref[...]

    return pl.pallas_call(
        body, out_shape=jax.ShapeDtypeStruct((N_PAD, D2P), jnp.float32))(
            S1, g1, dinv_col, b1_row, W2p)


def _tc3(S2, g2, dinv_col, b2_row):
    def body(s_ref, g2_ref, dv_ref, b2_ref, out_ref):
        agg = (s_ref[0] + s_ref[1] + g2_ref[...]) * dv_ref[...]
        z = agg[:, :D2] + b2_ref[...]
        m = jnp.max(z, axis=1, keepdims=True)
        lse = jnp.log(jnp.sum(jnp.exp(z - m), axis=1, keepdims=True)) + m
        out_ref[...] = z - lse

    return pl.pallas_call(
        body, out_shape=jax.ShapeDtypeStruct((N_PAD, D2), jnp.float32))(
            S2, g2, dinv_col, b2_row)


def kernel(x, edge_index, W1, b1, W2, b2):
    ei = edge_index.astype(jnp.int32)
    pad = jnp.full((E_PAD - E,), N, jnp.int32)
    row = jnp.concatenate([ei[0], pad])
    col = jnp.concatenate([ei[1], pad])
    row3 = row.reshape(NW, NCHK, CH)
    col3 = col.reshape(NW, NCHK, CH)
    col2 = col.reshape(NW, E_PW)
    x_pad = jnp.pad(x, ((0, N_PAD - N), (0, 0)))
    W2p = jnp.pad(W2, ((0, 0), (0, D2P - D2)))
    zeros16 = jnp.zeros((N_PAD, D1), jnp.float32)
    zeros48 = jnp.zeros((N_PAD, D2P), jnp.float32)

    degp = _sc_deg(col2)
    dinv_row = _tc_deg(degp)
    dinv_col = dinv_row.reshape(N_PAD, 1)
    g1 = _tc1(x_pad, W1, dinv_col)
    S1 = _sc_scatter16(g1, row3, col3, zeros16)
    g2 = _tc2(S1, g1, dinv_col, b1.reshape(1, D1), W2p)
    S2 = _sc_scatter48(g2, row3, col3, zeros48)
    out = _tc3(S2, g2, dinv_col, b2.reshape(1, D2))
    return out[:N]
